# baseline (speedup 1.0000x reference)
import functools

import jax
import jax.numpy as jnp
from jax import lax
from jax.experimental import pallas as pl
from jax.experimental.pallas import tpu as pltpu
from jax.experimental.pallas import tpu_sc as plsc

B = 4096
N = 200
HID = 64
R = B * N
CLIP = 1000
HALF = CLIP // 2
NT = 13
NC, NS, L = 2, 16, 16
NW = NC * NS
ROWS_PER_W = R // NW
CH = 128
N_CHUNKS = ROWS_PER_W // CH


def _trunc_div2(t):
    return jnp.where(t < 0, t + 1, t) >> 1


def _clip(v):
    return jnp.minimum(jnp.maximum(v, 0), CLIP)


@functools.partial(
    pl.kernel,
    out_type=jax.ShapeDtypeStruct((R, HID), jnp.float32),
    mesh=plsc.VectorSubcoreMesh(core_axis_name="c", subcore_axis_name="s"),
    compiler_params=pltpu.CompilerParams(use_tc_tiling_on_sc=False,
                                         needs_layout_passes=False),
    scratch_types=[
        pltpu.VMEM((2, 7, CH), jnp.int32),
        pltpu.VMEM((2, NT * CH), jnp.int32),
        pltpu.VMEM((2, NT * CH, HID), jnp.bfloat16),
        pltpu.VMEM((2, CH, HID), jnp.float32),
        pltpu.SemaphoreType.DMA,
        pltpu.SemaphoreType.DMA,
        pltpu.SemaphoreType.DMA,
        pltpu.SemaphoreType.DMA,
        pltpu.SemaphoreType.DMA,
        pltpu.SemaphoreType.DMA,
    ],
)
def _gather_sum(bt0, bt1, bt2, bt3, bt4, bt5, bt6, wall, out,
                bx_v, idx_v, buf_v, acc_v,
                gsem0, gsem1, bxsem0, bxsem1, osem0, osem1):
    bts = (bt0, bt1, bt2, bt3, bt4, bt5, bt6)
    gsem = (gsem0, gsem1)
    bxsem = (bxsem0, bxsem1)
    osem = (osem0, osem1)
    wid = lax.axis_index("s") * NC + lax.axis_index("c")
    w_base = wid * ROWS_PER_W
    roff = (wid & 15) * (NT * 1024)

    def compute_idx(slot):
        for g in range(CH // L):
            sl = pl.ds(g * L, L)
            cx = bx_v[slot, 0, sl]
            cy = bx_v[slot, 1, sl]
            w = bx_v[slot, 2, sl]
            h = bx_v[slot, 3, sl]
            xs = bx_v[slot, 4, sl]
            ys = bx_v[slot, 5, sl]
            lab = bx_v[slot, 6, sl]
            xa = _trunc_div2(xs - HALF)
            ya = _trunc_div2(ys - HALF)
            wh = w >> 1
            hh = h >> 1
            idx_v[slot, pl.ds(0 * CH + g * L, L)] = w + roff
            idx_v[slot, pl.ds(1 * CH + g * L, L)] = h + 1024
            idx_v[slot, pl.ds(2 * CH + g * L, L)] = cx + (roff + 2 * 1024)
            idx_v[slot, pl.ds(3 * CH + g * L, L)] = cy + (roff + 3 * 1024)
            idx_v[slot, pl.ds(4 * CH + g * L, L)] = xs + (roff + 4 * 1024)
            idx_v[slot, pl.ds(5 * CH + g * L, L)] = ys + (roff + 5 * 1024)
            idx_v[slot, pl.ds(6 * CH + g * L, L)] = lab + (roff + 6 * 1024)
            idx_v[slot, pl.ds(7 * CH + g * L, L)] = _clip(cx - wh - xa) + (roff + 7 * 1024)
            idx_v[slot, pl.ds(8 * CH + g * L, L)] = _clip(cx + wh - xa) + (roff + 8 * 1024)
            idx_v[slot, pl.ds(9 * CH + g * L, L)] = _clip(cx + wh + xa) + (roff + 9 * 1024)
            idx_v[slot, pl.ds(10 * CH + g * L, L)] = _clip(cx - wh + xa) + (roff + 10 * 1024)
            idx_v[slot, pl.ds(11 * CH + g * L, L)] = _clip(cy - hh - ya) + (roff + 11 * 1024)
            idx_v[slot, pl.ds(12 * CH + g * L, L)] = _clip(cy + hh + ya) + (roff + 12 * 1024)

    def fire_gather(slot):
        pltpu.async_copy(wall.at[idx_v.at[slot]], buf_v.at[slot], gsem[slot])

    def wait_gather(slot):
        pltpu.make_async_copy(wall.at[idx_v.at[slot]], buf_v.at[slot],
                              gsem[slot]).wait()

    def fire_bx(c, plane):
        base = w_base + c * CH
        for comp in range(7):
            pltpu.async_copy(bts[comp].at[pl.ds(base, CH)],
                             bx_v.at[plane, comp], bxsem[plane])

    def wait_bx(plane):
        for comp in range(7):
            pltpu.make_async_copy(bts[comp].at[pl.ds(0, CH)],
                                  bx_v.at[plane, comp], bxsem[plane]).wait()

    def sum_and_emit(slot, c):
        def sum_row(r2, carry):
            for rr in range(4):
                r = r2 * 4 + rr
                for g2 in range(HID // 32):
                    sl32 = pl.ds(g2 * 32, 32)
                    vals = [buf_v[slot, t * CH + r, sl32] for t in range(NT)]
                    while len(vals) > 1:
                        nxt = [vals[i] + vals[i + 1]
                               for i in range(0, len(vals) - 1, 2)]
                        if len(vals) % 2:
                            nxt.append(vals[-1])
                        vals = nxt
                    a, bb = plsc.unpack(vals[0],
                                        format=plsc.PackFormat.INTERLEAVED)
                    acc_v[slot, r, pl.ds(g2 * 32, L)] = a
                    acc_v[slot, r, pl.ds(g2 * 32 + L, L)] = bb
            return carry

        lax.fori_loop(0, CH // 4, sum_row, 0)
        pltpu.async_copy(acc_v.at[slot],
                         out.at[pl.ds(w_base + c * CH, CH)], osem[slot])

    def drain_out(slot):
        pltpu.make_async_copy(acc_v.at[slot], out.at[pl.ds(0, CH)],
                              osem[slot]).wait()

    fire_bx(0, 0)
    wait_bx(0)
    compute_idx(0)
    fire_gather(0)
    fire_bx(1, 1)

    def body(i2, carry):
        for p in (0, 1):
            c = 2 * i2 + p
            q = 1 - p

            @pl.when(c + 1 < N_CHUNKS)
            def _(q=q):
                wait_bx(q)
                compute_idx(q)
                fire_gather(q)

            @pl.when(c + 2 < N_CHUNKS)
            def _(c=c, p=p):
                fire_bx(c + 2, p)

            wait_gather(p)

            @pl.when(c >= 2)
            def _(p=p):
                drain_out(p)

            sum_and_emit(p, c)
        return carry

    lax.fori_loop(0, N_CHUNKS // 2, body, 0)
    drain_out(0)
    drain_out(1)


def kernel(boxes, input_box_counts, W_w, W_h, W_cx, W_cy, W_xskew, W_yskew,
           W_label, W_x1, W_y1, W_x2, W_y2, W_x3, W_y3, W_x4, W_y4):
    del input_box_counts
    comps = [boxes[:, :, c].reshape(R) for c in range(7)]

    def prep(w):
        v = w.shape[0]
        w = w.reshape(v, HID // 32, 2, L).swapaxes(2, 3).reshape(v, HID)
        return w.astype(jnp.bfloat16)

    wall = jnp.concatenate(
        [prep(w) for w in
         (W_w, W_h, W_cx, W_cy, W_xskew, W_yskew, W_label,
          W_x1, W_x2, W_x3, W_x4, W_y1 + W_y4, W_y2 + W_y3)], axis=0)
    wall = jnp.tile(wall, (16, 1))
    return _gather_sum(*comps, wall).reshape(B, N, HID)


# speedup vs baseline: 1.0017x; 1.0017x over previous
import functools

import jax
import jax.numpy as jnp
from jax import lax
from jax.experimental import pallas as pl
from jax.experimental.pallas import tpu as pltpu
from jax.experimental.pallas import tpu_sc as plsc

B = 4096
N = 200
HID = 64
R = B * N
CLIP = 1000
HALF = CLIP // 2
NT = 13
NC, NS, L = 2, 16, 16
NW = NC * NS
ROWS_PER_W = R // NW
CH = 128
N_CHUNKS = ROWS_PER_W // CH


def _trunc_div2(t):
    return jnp.where(t < 0, t + 1, t) >> 1


def _clip(v):
    return jnp.minimum(jnp.maximum(v, 0), CLIP)


@functools.partial(
    pl.kernel,
    out_type=jax.ShapeDtypeStruct((R, HID), jnp.float32),
    mesh=plsc.VectorSubcoreMesh(core_axis_name="c", subcore_axis_name="s"),
    compiler_params=pltpu.CompilerParams(use_tc_tiling_on_sc=False,
                                         needs_layout_passes=False),
    scratch_types=[
        pltpu.VMEM((2, 7, CH), jnp.int32),
        pltpu.VMEM((2, NT * CH), jnp.int32),
        pltpu.VMEM((2, NT * CH, HID), jnp.bfloat16),
        pltpu.VMEM((2, CH, HID), jnp.float32),
        pltpu.SemaphoreType.DMA,
        pltpu.SemaphoreType.DMA,
        pltpu.SemaphoreType.DMA,
        pltpu.SemaphoreType.DMA,
        pltpu.SemaphoreType.DMA,
        pltpu.SemaphoreType.DMA,
    ],
)
def _gather_sum(bt0, bt1, bt2, bt3, bt4, bt5, bt6, wall, out,
                bx_v, idx_v, buf_v, acc_v,
                gsem0, gsem1, bxsem0, bxsem1, osem0, osem1):
    bts = (bt0, bt1, bt2, bt3, bt4, bt5, bt6)
    gsem = (gsem0, gsem1)
    bxsem = (bxsem0, bxsem1)
    osem = (osem0, osem1)
    wid = lax.axis_index("s") * NC + lax.axis_index("c")
    w_base = wid * ROWS_PER_W
    roff = (wid & 15) * (NT * 1024)

    def compute_idx(slot):
        for g in range(CH // L):
            sl = pl.ds(g * L, L)
            cx = bx_v[slot, 0, sl]
            cy = bx_v[slot, 1, sl]
            w = bx_v[slot, 2, sl]
            h = bx_v[slot, 3, sl]
            xs = bx_v[slot, 4, sl]
            ys = bx_v[slot, 5, sl]
            lab = bx_v[slot, 6, sl]
            xa = _trunc_div2(xs - HALF)
            ya = _trunc_div2(ys - HALF)
            wh = w >> 1
            hh = h >> 1
            idx_v[slot, pl.ds(0 * CH + g * L, L)] = w + roff
            idx_v[slot, pl.ds(1 * CH + g * L, L)] = h + (roff + 1024)
            idx_v[slot, pl.ds(2 * CH + g * L, L)] = cx + (roff + 2 * 1024)
            idx_v[slot, pl.ds(3 * CH + g * L, L)] = cy + (roff + 3 * 1024)
            idx_v[slot, pl.ds(4 * CH + g * L, L)] = xs + (roff + 4 * 1024)
            idx_v[slot, pl.ds(5 * CH + g * L, L)] = ys + (roff + 5 * 1024)
            idx_v[slot, pl.ds(6 * CH + g * L, L)] = lab + (roff + 6 * 1024)
            idx_v[slot, pl.ds(7 * CH + g * L, L)] = _clip(cx - wh - xa) + (roff + 7 * 1024)
            idx_v[slot, pl.ds(8 * CH + g * L, L)] = _clip(cx + wh - xa) + (roff + 8 * 1024)
            idx_v[slot, pl.ds(9 * CH + g * L, L)] = _clip(cx + wh + xa) + (roff + 9 * 1024)
            idx_v[slot, pl.ds(10 * CH + g * L, L)] = _clip(cx - wh + xa) + (roff + 10 * 1024)
            idx_v[slot, pl.ds(11 * CH + g * L, L)] = _clip(cy - hh - ya) + (roff + 11 * 1024)
            idx_v[slot, pl.ds(12 * CH + g * L, L)] = _clip(cy + hh + ya) + (roff + 12 * 1024)

    def fire_gather(slot):
        pltpu.async_copy(wall.at[idx_v.at[slot]], buf_v.at[slot], gsem[slot])

    def wait_gather(slot):
        pltpu.make_async_copy(wall.at[idx_v.at[slot]], buf_v.at[slot],
                              gsem[slot]).wait()

    def fire_bx(c, plane):
        base = w_base + c * CH
        for comp in range(7):
            pltpu.async_copy(bts[comp].at[pl.ds(base, CH)],
                             bx_v.at[plane, comp], bxsem[plane])

    def wait_bx(plane):
        for comp in range(7):
            pltpu.make_async_copy(bts[comp].at[pl.ds(0, CH)],
                                  bx_v.at[plane, comp], bxsem[plane]).wait()

    def sum_and_emit(slot, c):
        def sum_row(r2, carry):
            for rr in range(2):
                r = r2 * 2 + rr
                for g2 in range(HID // 32):
                    sl32 = pl.ds(g2 * 32, 32)
                    vals = [buf_v[slot, t * CH + r, sl32] for t in range(NT)]
                    while len(vals) > 1:
                        nxt = [vals[i] + vals[i + 1]
                               for i in range(0, len(vals) - 1, 2)]
                        if len(vals) % 2:
                            nxt.append(vals[-1])
                        vals = nxt
                    a, bb = plsc.unpack(vals[0],
                                        format=plsc.PackFormat.INTERLEAVED)
                    acc_v[slot, r, pl.ds(g2 * 32, L)] = a
                    acc_v[slot, r, pl.ds(g2 * 32 + L, L)] = bb
            return carry

        lax.fori_loop(0, CH // 2, sum_row, 0)
        pltpu.async_copy(acc_v.at[slot],
                         out.at[pl.ds(w_base + c * CH, CH)], osem[slot])

    def drain_out(slot):
        pltpu.make_async_copy(acc_v.at[slot], out.at[pl.ds(0, CH)],
                              osem[slot]).wait()

    fire_bx(0, 0)
    wait_bx(0)
    compute_idx(0)
    fire_gather(0)
    fire_bx(1, 1)

    def body(i2, carry):
        for p in (0, 1):
            c = 2 * i2 + p
            q = 1 - p

            @pl.when(c + 1 < N_CHUNKS)
            def _(q=q):
                wait_bx(q)
                compute_idx(q)
                fire_gather(q)

            @pl.when(c + 2 < N_CHUNKS)
            def _(c=c, p=p):
                fire_bx(c + 2, p)

            wait_gather(p)

            @pl.when(c >= 2)
            def _(p=p):
                drain_out(p)

            sum_and_emit(p, c)
        return carry

    lax.fori_loop(0, N_CHUNKS // 2, body, 0)
    drain_out(0)
    drain_out(1)


def kernel(boxes, input_box_counts, W_w, W_h, W_cx, W_cy, W_xskew, W_yskew,
           W_label, W_x1, W_y1, W_x2, W_y2, W_x3, W_y3, W_x4, W_y4):
    del input_box_counts
    comps = [boxes[:, :, c].reshape(R) for c in range(7)]

    def prep(w):
        v = w.shape[0]
        w = w.reshape(v, HID // 32, 2, L).swapaxes(2, 3).reshape(v, HID)
        return w.astype(jnp.bfloat16)

    wall = jnp.concatenate(
        [prep(w) for w in
         (W_w, W_h, W_cx, W_cy, W_xskew, W_yskew, W_label,
          W_x1, W_x2, W_x3, W_x4, W_y1 + W_y4, W_y2 + W_y3)], axis=0)
    wall = jnp.tile(wall, (16, 1))
    return _gather_sum(*comps, wall).reshape(B, N, HID)
